# baseline (speedup 1.0000x reference)
import functools

import jax
import jax.numpy as jnp
from jax import lax
from jax.experimental import pallas as pl
from jax.experimental.pallas import tpu as pltpu
from jax.experimental.pallas import tpu_sc as plsc

BATCH = 1024
NUM_NUMERIC = 13
NUM_CAT = 26
DEPTH = 1001
OUT_COLS = NUM_CAT * DEPTH + NUM_NUMERIC

NUM_WORKERS = 32
ROWS_PER_W = BATCH // NUM_WORKERS
NBUF = 4
ROW_PAD = 26048


def _sc_body(cat_hbm, num_hbm, scale_hbm, shift_hbm, out_hbm,
             buf0, buf1, buf2, buf3, catv, numv, scalev, shiftv,
             sem0, sem1, sem2, sem3):
    wid = lax.axis_index("s") * 2 + lax.axis_index("c")
    base = wid * ROWS_PER_W

    pltpu.sync_copy(cat_hbm.at[pl.ds(base * 32, ROWS_PER_W * 32)], catv)
    pltpu.sync_copy(num_hbm.at[pl.ds(base * 16, ROWS_PER_W * 16)], numv)
    pltpu.sync_copy(scale_hbm, scalev)
    pltpu.sync_copy(shift_hbm, shiftv)

    iota = lax.broadcasted_iota(jnp.int32, (16,), 0)
    zeros16 = jnp.zeros((16,), jnp.float32)
    ones16 = jnp.ones((16,), jnp.float32)
    scale = scalev[...]
    shift = shiftv[...]

    bufs = (buf0, buf1, buf2, buf3)
    sems = (sem0, sem1, sem2, sem3)

    for buf in bufs:
        def zi(k, carry, buf=buf):
            buf[pl.ds(k * 16, 16)] = zeros16
            return carry
        lax.fori_loop(0, ROW_PAD // 16, zi, 0)

    def touch_row(buf, lr, clear):
        tlo = catv[pl.ds(lr * 32, 16)] + DEPTH * iota
        thi = catv[pl.ds(lr * 32 + 16, 16)] + DEPTH * (iota + 16)
        plsc.store_scatter(buf, [tlo], zeros16 if clear else ones16)
        plsc.store_scatter(buf, [thi], zeros16 if clear else ones16,
                           mask=iota < NUM_CAT - 16)
        if clear:
            nrm = zeros16
        else:
            nrm = numv[pl.ds(lr * 16, 16)] * scale - shift
        plsc.store_scatter(buf, [NUM_CAT * DEPTH + iota], nrm,
                           mask=iota < NUM_NUMERIC)

    copies = [None] * ROWS_PER_W
    for i in range(ROWS_PER_W):
        slot = i % NBUF
        buf, sem = bufs[slot], sems[slot]
        if i >= NBUF:
            copies[i - NBUF].wait()
            touch_row(buf, i - NBUF, clear=True)
        touch_row(buf, i, clear=False)
        copies[i] = pltpu.make_async_copy(
            buf.at[pl.ds(0, OUT_COLS)], out_hbm.at[base + i], sem)
        copies[i].start()
    for i in range(ROWS_PER_W - NBUF, ROWS_PER_W):
        copies[i].wait()


def kernel(numeric, cat_idx, mean, var):
    scale = 1.0 / jnp.maximum(jnp.sqrt(var), 1e-7)
    shift = mean * scale
    scale16 = jnp.pad(scale, (0, 16 - NUM_NUMERIC))
    shift16 = jnp.pad(shift, (0, 16 - NUM_NUMERIC))
    cat_flat = jnp.pad(cat_idx, ((0, 0), (0, 32 - NUM_CAT))).reshape(-1)
    num_flat = jnp.pad(numeric, ((0, 0), (0, 16 - NUM_NUMERIC))).reshape(-1)

    mesh = plsc.VectorSubcoreMesh(core_axis_name="c", subcore_axis_name="s")
    run = functools.partial(
        pl.kernel,
        out_type=jax.ShapeDtypeStruct((BATCH, OUT_COLS), jnp.float32),
        mesh=mesh,
        compiler_params=pltpu.CompilerParams(
            use_tc_tiling_on_sc=False, needs_layout_passes=False
        ),
        scratch_types=[
            pltpu.VMEM((ROW_PAD,), jnp.float32),
            pltpu.VMEM((ROW_PAD,), jnp.float32),
            pltpu.VMEM((ROW_PAD,), jnp.float32),
            pltpu.VMEM((ROW_PAD,), jnp.float32),
            pltpu.VMEM((ROWS_PER_W * 32,), jnp.int32),
            pltpu.VMEM((ROWS_PER_W * 16,), jnp.float32),
            pltpu.VMEM((16,), jnp.float32),
            pltpu.VMEM((16,), jnp.float32),
            pltpu.SemaphoreType.DMA,
            pltpu.SemaphoreType.DMA,
            pltpu.SemaphoreType.DMA,
            pltpu.SemaphoreType.DMA,
        ],
    )(_sc_body)
    return run(cat_flat, num_flat, scale16, shift16)


# speedup vs baseline: 2.2522x; 2.2522x over previous
import jax
import jax.numpy as jnp
from jax.experimental import pallas as pl

BATCH = 1024
NUM_NUMERIC = 13
NUM_CAT = 26
DEPTH = 1001
OUT_COLS = NUM_CAT * DEPTH + NUM_NUMERIC

BLOCK = 256


def _body(num_ref, cat_ref, mean_ref, var_ref, out_ref):
    iota = jax.lax.broadcasted_iota(jnp.int32, (BLOCK, DEPTH), 1)
    for f in range(NUM_CAT):
        sel = cat_ref[:, f : f + 1]
        out_ref[:, f * DEPTH : (f + 1) * DEPTH] = (iota == sel).astype(jnp.float32)
    inv = 1.0 / jnp.maximum(jnp.sqrt(var_ref[...]), 1e-7)
    out_ref[:, NUM_CAT * DEPTH :] = (num_ref[...] - mean_ref[...]) * inv


def kernel(numeric, cat_idx, mean, var):
    grid = (BATCH // BLOCK,)
    return pl.pallas_call(
        _body,
        grid=grid,
        in_specs=[
            pl.BlockSpec((BLOCK, NUM_NUMERIC), lambda i: (i, 0)),
            pl.BlockSpec((BLOCK, NUM_CAT), lambda i: (i, 0)),
            pl.BlockSpec((1, NUM_NUMERIC), lambda i: (0, 0)),
            pl.BlockSpec((1, NUM_NUMERIC), lambda i: (0, 0)),
        ],
        out_specs=pl.BlockSpec((BLOCK, OUT_COLS), lambda i: (i, 0)),
        out_shape=jax.ShapeDtypeStruct((BATCH, OUT_COLS), jnp.float32),
    )(numeric, cat_idx, mean.reshape(1, -1), var.reshape(1, -1))
